# baseline (speedup 1.0000x reference)
import jax
import jax.numpy as jnp
from jax import lax
from jax.experimental import pallas as pl
from jax.experimental.pallas import tpu as pltpu

D = 768
E = 64
K = 2
DFF = 128
T = 2048
NUM_ITERS = 2
MIN_ENT = 0.8
EC = 8
NPH = 1 + E // EC


def _mm(a, b):
    return jax.lax.dot_general(a, b, (((1,), (0,)), ((), ())),
                               preferred_element_type=jnp.float32)


def _fused_kernel(x_ref, wr_ref, br_ref, noise_ref, w1_ref, b1_ref, w2_ref,
                  b2_ref, logits_ref, usage_ref, lb_ref, states_ref,
                  state_s, comb_s):
    g = pl.program_id(0)
    phase = g % NPH

    @pl.when(g == 0)
    def _init():
        state_s[...] = x_ref[...]

    @pl.when(phase == 0)
    def _router():
        state = state_s[...]
        logits = _mm(state, wr_ref[...]) + br_ref[...]
        m = jnp.max(logits, axis=-1, keepdims=True)
        ex = jnp.exp(logits - m)
        probs = ex / jnp.sum(ex, axis=-1, keepdims=True)
        entropy = jnp.mean(-jnp.sum(probs * jnp.log(probs), axis=-1))
        logits_ref[0] = jnp.where(entropy < MIN_ENT, logits + noise_ref[0],
                                  logits)
        iota = lax.broadcasted_iota(jnp.int32, (T, E), 1)
        w1v = jnp.max(probs, axis=-1, keepdims=True)
        i1 = jnp.min(jnp.where(probs == w1v, iota, E), axis=-1, keepdims=True)
        oh1 = iota == i1
        probs2 = jnp.where(oh1, -1.0, probs)
        w2v = jnp.max(probs2, axis=-1, keepdims=True)
        i2 = jnp.min(jnp.where(probs2 == w2v, iota, E), axis=-1,
                     keepdims=True)
        oh2 = iota == i2
        s = w1v + w2v
        combine = (jnp.where(oh1, w1v, 0.0) + jnp.where(oh2, w2v, 0.0)) / s
        comb_s[...] = combine
        counts = jnp.sum(oh1.astype(jnp.float32) + oh2.astype(jnp.float32),
                         axis=0, keepdims=True)
        usage_ref[0] = counts / T
        Pm = jnp.mean(probs, axis=0, keepdims=True)
        lb_ref[...] = jnp.sum((counts / (T * K)) * Pm).reshape(1, 1) * E
        states_ref[0] = state + _mm(combine, b2_ref[...])

    @pl.when(phase > 0)
    def _ffn():
        h = _mm(state_s[...].astype(jnp.bfloat16), w1_ref[...]) + b1_ref[...]
        a = jnp.maximum(h, 0.0)
        ch = phase - 1
        erow = lax.broadcasted_iota(jnp.int32, (E, EC * DFF), 0)
        ecol = lax.broadcasted_iota(jnp.int32, (E, EC * DFF), 1) // DFF
        expand = (erow == ecol + ch * EC).astype(jnp.float32)
        scale = _mm(comb_s[...], expand)
        states_ref[0] += _mm((a * scale).astype(jnp.bfloat16), w2_ref[...])

    @pl.when(phase == NPH - 1)
    def _carry():
        state_s[...] = states_ref[0]


def kernel(x, Wr, br, W1, b1, W2, b2):
    B, S, Dm = x.shape
    xs = x.reshape(T, D)
    W1c = W1.transpose(1, 0, 2).reshape(D, E * DFF).astype(jnp.bfloat16)
    W2c = W2.reshape(E * DFF, D).astype(jnp.bfloat16)
    b1c = b1.reshape(1, E * DFF)
    br2 = br.reshape(1, E)
    noise = jnp.stack([
        jax.random.normal(jax.random.fold_in(jax.random.key(1), it), (T, E),
                          dtype=jnp.float32) * 0.1
        for it in range(NUM_ITERS)])

    def _chunk(g):
        ph = g % NPH
        return jnp.maximum(ph - 1, 0)

    logits, usage, lb, states = pl.pallas_call(
        _fused_kernel,
        grid=(NUM_ITERS * NPH,),
        in_specs=[
            pl.BlockSpec((T, D), lambda g: (0, 0)),
            pl.BlockSpec((D, E), lambda g: (0, 0)),
            pl.BlockSpec((1, E), lambda g: (0, 0)),
            pl.BlockSpec((1, T, E), lambda g: (g // NPH, 0, 0)),
            pl.BlockSpec((D, EC * DFF), lambda g: (0, _chunk(g))),
            pl.BlockSpec((1, EC * DFF), lambda g: (0, _chunk(g))),
            pl.BlockSpec((EC * DFF, D), lambda g: (_chunk(g), 0)),
            pl.BlockSpec((E, D), lambda g: (0, 0)),
        ],
        out_specs=[
            pl.BlockSpec((1, T, E), lambda g: (g // NPH, 0, 0)),
            pl.BlockSpec((1, 1, E), lambda g: (g // NPH, 0, 0)),
            pl.BlockSpec((1, 1), lambda g: (0, 0)),
            pl.BlockSpec((1, T, D), lambda g: (g // NPH, 0, 0)),
        ],
        out_shape=[
            jax.ShapeDtypeStruct((NUM_ITERS, T, E), jnp.float32),
            jax.ShapeDtypeStruct((NUM_ITERS, 1, E), jnp.float32),
            jax.ShapeDtypeStruct((1, 1), jnp.float32),
            jax.ShapeDtypeStruct((NUM_ITERS, T, D), jnp.float32),
        ],
        scratch_shapes=[
            pltpu.VMEM((T, D), jnp.float32),
            pltpu.VMEM((T, E), jnp.float32),
        ],
        compiler_params=pltpu.CompilerParams(
            dimension_semantics=("arbitrary",)),
    )(xs, Wr, br2, noise, W1c, b1c, W2c, b2)

    final_output = states[NUM_ITERS - 1].reshape(B, S, Dm)
    return (final_output, lb.reshape(()), logits,
            usage.reshape(NUM_ITERS, E), states)


# speedup vs baseline: 1.0380x; 1.0380x over previous
import jax
import jax.numpy as jnp
from jax import lax
from jax.experimental import pallas as pl
from jax.experimental.pallas import tpu as pltpu

D = 768
E = 64
K = 2
DFF = 128
T = 2048
NUM_ITERS = 2
MIN_ENT = 0.8
EC = 8
NPH = 1 + E // EC


def _mm(a, b):
    return jax.lax.dot_general(a, b, (((1,), (0,)), ((), ())),
                               preferred_element_type=jnp.float32)


def _fused_kernel(x_ref, wr_ref, br_ref, noise_ref, w1_ref, b1_ref, w2_ref,
                  b2_ref, logits_ref, usage_ref, lb_ref, states_ref,
                  state_s, comb_s):
    g = pl.program_id(0)
    phase = g % NPH

    @pl.when(g == 0)
    def _init():
        state_s[...] = x_ref[...]

    @pl.when(phase == 0)
    def _router():
        state = state_s[...]
        logits = _mm(state, wr_ref[...]) + br_ref[...]
        m = jnp.max(logits, axis=-1, keepdims=True)
        ex = jnp.exp(logits - m)
        probs = ex / jnp.sum(ex, axis=-1, keepdims=True)
        entropy = jnp.mean(-jnp.sum(probs * jnp.log(probs), axis=-1))
        logits_ref[0] = jnp.where(entropy < MIN_ENT, logits + noise_ref[0],
                                  logits)
        iota = lax.broadcasted_iota(jnp.int32, (T, E), 1)
        w1v = jnp.max(probs, axis=-1, keepdims=True)
        i1 = jnp.min(jnp.where(probs == w1v, iota, E), axis=-1, keepdims=True)
        oh1 = iota == i1
        probs2 = jnp.where(oh1, -1.0, probs)
        w2v = jnp.max(probs2, axis=-1, keepdims=True)
        i2 = jnp.min(jnp.where(probs2 == w2v, iota, E), axis=-1,
                     keepdims=True)
        oh2 = iota == i2
        s = w1v + w2v
        combine = (jnp.where(oh1, w1v, 0.0) + jnp.where(oh2, w2v, 0.0)) / s
        comb_s[...] = combine
        counts = jnp.sum(oh1.astype(jnp.float32) + oh2.astype(jnp.float32),
                         axis=0, keepdims=True)
        usage_ref[0] = counts / T
        Pm = jnp.mean(probs, axis=0, keepdims=True)
        lb_ref[...] = jnp.sum((counts / (T * K)) * Pm).reshape(1, 1) * E
        states_ref[0] = state + _mm(combine, b2_ref[...])

    @pl.when(phase > 0)
    def _ffn():
        h = _mm(state_s[...], w1_ref[...]) + b1_ref[...]
        a = jnp.maximum(h, 0.0)
        ch = phase - 1
        erow = lax.broadcasted_iota(jnp.int32, (E, EC * DFF), 0)
        ecol = lax.broadcasted_iota(jnp.int32, (E, EC * DFF), 1) // DFF
        expand = (erow == ecol + ch * EC).astype(jnp.float32)
        scale = _mm(comb_s[...], expand)
        states_ref[0] += _mm(a * scale, w2_ref[...])

    @pl.when(phase == NPH - 1)
    def _carry():
        state_s[...] = states_ref[0]


def kernel(x, Wr, br, W1, b1, W2, b2):
    B, S, Dm = x.shape
    xs = x.reshape(T, D)
    W1c = W1.transpose(1, 0, 2).reshape(D, E * DFF)
    W2c = W2.reshape(E * DFF, D)
    b1c = b1.reshape(1, E * DFF)
    br2 = br.reshape(1, E)
    noise = jnp.stack([
        jax.random.normal(jax.random.fold_in(jax.random.key(1), it), (T, E),
                          dtype=jnp.float32) * 0.1
        for it in range(NUM_ITERS)])

    def _chunk(g):
        ph = g % NPH
        return jnp.maximum(ph - 1, 0)

    logits, usage, lb, states = pl.pallas_call(
        _fused_kernel,
        grid=(NUM_ITERS * NPH,),
        in_specs=[
            pl.BlockSpec((T, D), lambda g: (0, 0)),
            pl.BlockSpec((D, E), lambda g: (0, 0)),
            pl.BlockSpec((1, E), lambda g: (0, 0)),
            pl.BlockSpec((1, T, E), lambda g: (g // NPH, 0, 0)),
            pl.BlockSpec((D, EC * DFF), lambda g: (0, _chunk(g))),
            pl.BlockSpec((1, EC * DFF), lambda g: (0, _chunk(g))),
            pl.BlockSpec((EC * DFF, D), lambda g: (_chunk(g), 0)),
            pl.BlockSpec((E, D), lambda g: (0, 0)),
        ],
        out_specs=[
            pl.BlockSpec((1, T, E), lambda g: (g // NPH, 0, 0)),
            pl.BlockSpec((1, 1, E), lambda g: (g // NPH, 0, 0)),
            pl.BlockSpec((1, 1), lambda g: (0, 0)),
            pl.BlockSpec((1, T, D), lambda g: (g // NPH, 0, 0)),
        ],
        out_shape=[
            jax.ShapeDtypeStruct((NUM_ITERS, T, E), jnp.float32),
            jax.ShapeDtypeStruct((NUM_ITERS, 1, E), jnp.float32),
            jax.ShapeDtypeStruct((1, 1), jnp.float32),
            jax.ShapeDtypeStruct((NUM_ITERS, T, D), jnp.float32),
        ],
        scratch_shapes=[
            pltpu.VMEM((T, D), jnp.float32),
            pltpu.VMEM((T, E), jnp.float32),
        ],
        compiler_params=pltpu.CompilerParams(
            dimension_semantics=("arbitrary",)),
    )(xs, Wr, br2, noise, W1c, b1c, W2c, b2)

    final_output = states[NUM_ITERS - 1].reshape(B, S, Dm)
    return (final_output, lb.reshape(()), logits,
            usage.reshape(NUM_ITERS, E), states)
